# Initial kernel scaffold; baseline (speedup 1.0000x reference)
import jax
import jax.numpy as jnp
from jax.experimental import pallas as pl


def kernel(x_dyn, x_topo, edge_weight, params, edge_index, batch):
    raise NotImplementedError("write your pallas kernel here")


# speedup vs baseline: 17.8866x; 17.8866x over previous
import functools
import numpy as np

import jax
import jax.numpy as jnp
from jax import lax
from jax.experimental import pallas as pl
from jax.experimental.pallas import tpu as pltpu
from jax.experimental.pallas import tpu_sc as plsc

N = 10000
NP = 10240
E = 320000
D = 128
S = 5
NH = 4
HD = 32
TNL = 3
FF = 512

NC, NS, L = 2, 16, 16
CH = 128
ET = E // NS
NCHUNK = (ET + CH - 1) // CH
EP = NCHUNK * CH
ACC_ROWS = 20736
PT = ACC_ROWS // NS
TRASH = 2 * NP

BN = 512


def _ln(x, g, b, eps=1e-5):
    m = jnp.mean(x, axis=-1, keepdims=True)
    v = jnp.mean((x - m) ** 2, axis=-1, keepdims=True)
    return (x - m) * lax.rsqrt(v + eps) * g + b


def _dot(a, b):
    return jax.lax.dot_general(a, b, (((1,), (0,)), ((), ())),
                               preferred_element_type=jnp.float32)


def _edgeprep_body(row_r, col_r, ew_r, g0_r, g1_r, tc_r):
    row = row_r[...]
    col = col_r[...]
    ew = ew_r[...]
    sgn = (ew < 0).astype(jnp.int32)
    nz = ew != 0
    g0_r[...] = jnp.where(nz, (2 * sgn + 0) * NP + row, 0)
    g1_r[...] = jnp.where(nz, (2 * sgn + 1) * NP + row, 0)
    tc_r[...] = jnp.where(nz, sgn * NP + col, TRASH)


def _edgeprep(row, col, ew):
    spec_i = pl.BlockSpec((2500, 128), lambda: (0, 0))
    return pl.pallas_call(
        _edgeprep_body,
        in_specs=[spec_i, spec_i, spec_i],
        out_specs=[spec_i, spec_i, spec_i],
        out_shape=[jax.ShapeDtypeStruct((2500, 128), jnp.int32)] * 3,
    )(row, col, ew)


def _degree_sc(tcol_sc):
    mesh = plsc.VectorSubcoreMesh(core_axis_name="c", subcore_axis_name="s")

    @functools.partial(
        pl.kernel,
        out_type=jax.ShapeDtypeStruct((NC * ACC_ROWS,), jnp.float32),
        mesh=mesh,
        scratch_types=[
            pltpu.VMEM((NCHUNK, CH), jnp.int32),
            pltpu.VMEM((CH,), jnp.float32),
            pltpu.VMEM((PT,), jnp.float32),
            pltpu.VMEM_SHARED((ACC_ROWS,), jnp.float32),
            pltpu.SemaphoreType.DMA,
        ],
    )
    def deg_kernel(tcol_hbm, out_hbm, tcol_v, ones_v, zbuf, acc, sem):
        c = lax.axis_index("c")
        s = lax.axis_index("s")
        for k in range(CH // L):
            ones_v[pl.ds(16 * k, 16)] = jnp.ones((16,), jnp.float32)
        for k in range(PT // L):
            zbuf[pl.ds(16 * k, 16)] = jnp.zeros((16,), jnp.float32)
        base = pl.multiple_of(s * PT, 8)
        pltpu.sync_copy(zbuf, acc.at[pl.ds(base, PT)])
        plsc.subcore_barrier()
        pltpu.sync_copy(tcol_hbm.at[s], tcol_v)

        def body(j, carry):
            pltpu.sync_copy(ones_v, acc.at[tcol_v.at[j]], add=True)
            return carry

        lax.fori_loop(0, NCHUNK, body, 0)
        plsc.subcore_barrier()
        obase = pl.multiple_of(c * ACC_ROWS + s * PT, 8)
        pltpu.sync_copy(acc.at[pl.ds(base, PT)], zbuf)
        pltpu.sync_copy(zbuf, out_hbm.at[pl.ds(obase, PT)])

    return deg_kernel(tcol_sc).reshape(NC, ACC_ROWS)


def _agg_sc(hs_flat, gidx_sc, tcol_sc):
    mesh = plsc.VectorSubcoreMesh(core_axis_name="c", subcore_axis_name="s")

    @functools.partial(
        pl.kernel,
        out_type=jax.ShapeDtypeStruct((NC, ACC_ROWS, 64), jnp.float32),
        mesh=mesh,
        scratch_types=[
            pltpu.VMEM((32, CH), jnp.int32),
            pltpu.VMEM((32, CH), jnp.int32),
            pltpu.VMEM((CH, 64), jnp.float32),
            pltpu.VMEM((144, 64), jnp.float32),
            pltpu.VMEM_SHARED((ACC_ROWS, 64), jnp.float32),
            pltpu.SemaphoreType.DMA,
        ],
        compiler_params=pltpu.CompilerParams(use_tc_tiling_on_sc=False),
    )
    def agg_kernel(hs_hbm, gidx_hbm, tcol_hbm, out_hbm,
                   gidx_v, tcol_v, rows_v, zbuf, acc, sem):
        c = lax.axis_index("c")
        s = lax.axis_index("s")
        for r in range(144):
            for k in range(64 // L):
                zbuf[r, pl.ds(16 * k, 16)] = jnp.zeros((16,), jnp.float32)
        base = pl.multiple_of(s * PT, 8)
        for k in range(PT // 144):
            pltpu.sync_copy(zbuf, acc.at[pl.ds(base + k * 144, 144)])
        plsc.subcore_barrier()

        def body(j, carry):
            pltpu.async_copy(hs_hbm.at[gidx_v.at[j]], rows_v, sem).wait()
            pltpu.sync_copy(rows_v, acc.at[tcol_v.at[j]], add=True)
            return carry

        for blk in range(5):
            nch = min(32, NCHUNK - blk * 32)
            pltpu.sync_copy(gidx_hbm.at[c, s, pl.ds(blk * 32, nch)],
                            gidx_v.at[pl.ds(0, nch)])
            pltpu.sync_copy(tcol_hbm.at[s, pl.ds(blk * 32, nch)],
                            tcol_v.at[pl.ds(0, nch)])
            lax.fori_loop(0, nch, body, 0)
        plsc.subcore_barrier()
        for k in range(PT // 144):
            pltpu.sync_copy(acc.at[pl.ds(base + k * 144, 144)], zbuf)
            pltpu.sync_copy(zbuf, out_hbm.at[c].at[pl.ds(base + k * 144, 144)])

    return agg_kernel(hs_flat, gidx_sc, tcol_sc)


def _dinv_body(deg_r, dinv_r, dsq_r):
    deg = deg_r[...] + 1.0
    dinv_r[...] = lax.rsqrt(deg)
    dsq_r[...] = 1.0 / deg


def _dinv_tc(deg):
    spec = pl.BlockSpec((162, 128), lambda: (0, 0))
    return pl.pallas_call(
        _dinv_body,
        in_specs=[spec],
        out_specs=[spec, spec],
        out_shape=[jax.ShapeDtypeStruct((162, 128), jnp.float32)] * 2,
    )(deg)


def _hsbuild_body(h_r, dp_r, dn_r, hs_r):
    h = h_r[...]
    dp = dp_r[...]
    dn = dn_r[...]
    hs_r[0] = dp * h[:, :64]
    hs_r[1] = dp * h[:, 64:]
    hs_r[2] = dn * h[:, :64]
    hs_r[3] = dn * h[:, 64:]


def _hsbuild(h, dinvp, dinvn):
    grid = NP // BN
    return pl.pallas_call(
        _hsbuild_body,
        grid=(grid,),
        in_specs=[
            pl.BlockSpec((BN, 128), lambda i: (i, 0)),
            pl.BlockSpec((BN, 1), lambda i: (i, 0)),
            pl.BlockSpec((BN, 1), lambda i: (i, 0)),
        ],
        out_specs=pl.BlockSpec((4, BN, 64), lambda i: (0, i, 0)),
        out_shape=jax.ShapeDtypeStruct((4, NP, 64), jnp.float32),
    )(h, dinvp, dinvn)


def _layer_body(h_r, a0p_r, a0n_r, a1p_r, a1n_r, dp_r, dn_r, d2p_r, d2n_r,
                pwT_r, pb_r, nwT_r, nb_r, hn_r, hs_r):
    h = h_r[...]
    dp = dp_r[...]
    dn = dn_r[...]
    ap = dp * jnp.concatenate([a0p_r[...], a1p_r[...]], axis=1) + d2p_r[...] * h
    an = dn * jnp.concatenate([a0n_r[...], a1n_r[...]], axis=1) + d2n_r[...] * h
    px = _dot(ap, pwT_r[...]) + pb_r[...]
    nx = _dot(an, nwT_r[...]) + nb_r[...]
    hn = jax.nn.relu(px) - jax.nn.relu(nx)
    hn_r[...] = hn
    hs_r[0] = dp * hn[:, :64]
    hs_r[1] = dp * hn[:, 64:]
    hs_r[2] = dn * hn[:, :64]
    hs_r[3] = dn * hn[:, 64:]


def _layer_last_body(h_r, a0p_r, a0n_r, a1p_r, a1n_r, dp_r, dn_r, d2p_r,
                     d2n_r, pwT_r, pb_r, nwT_r, nb_r, sum_r):
    i = pl.program_id(0)
    h = h_r[...]
    ap = dp_r[...] * jnp.concatenate([a0p_r[...], a1p_r[...]], axis=1) \
        + d2p_r[...] * h
    an = dn_r[...] * jnp.concatenate([a0n_r[...], a1n_r[...]], axis=1) \
        + d2n_r[...] * h
    px = _dot(ap, pwT_r[...]) + pb_r[...]
    nx = _dot(an, nwT_r[...]) + nb_r[...]
    hn = jax.nn.relu(px) - jax.nn.relu(nx)
    ridx = lax.broadcasted_iota(jnp.int32, (BN, 1), 0) + i * BN
    hn = jnp.where(ridx < N, hn, 0.0)
    part = jnp.sum(hn, axis=0, keepdims=True)
    prev = jnp.where(i == 0, jnp.zeros((1, D), jnp.float32), sum_r[...])
    sum_r[...] = prev + part


def _gcn_layer(h, agg, dinvp, dinvn, dp2, dn2, pwT, pb, nwT, nb, last):
    grid = NP // BN
    nblk = NP // BN
    a0 = agg[0]
    a1 = agg[1]
    in_specs = [
        pl.BlockSpec((BN, 128), lambda i: (i, 0)),
        pl.BlockSpec((BN, 64), lambda i: (i, 0)),
        pl.BlockSpec((BN, 64), lambda i: (i + nblk, 0)),
        pl.BlockSpec((BN, 64), lambda i: (i, 0)),
        pl.BlockSpec((BN, 64), lambda i: (i + nblk, 0)),
        pl.BlockSpec((BN, 1), lambda i: (i, 0)),
        pl.BlockSpec((BN, 1), lambda i: (i, 0)),
        pl.BlockSpec((BN, 1), lambda i: (i, 0)),
        pl.BlockSpec((BN, 1), lambda i: (i, 0)),
        pl.BlockSpec((128, 128), lambda i: (0, 0)),
        pl.BlockSpec((1, 128), lambda i: (0, 0)),
        pl.BlockSpec((128, 128), lambda i: (0, 0)),
        pl.BlockSpec((1, 128), lambda i: (0, 0)),
    ]
    args = (h, a0, a0, a1, a1, dinvp, dinvn, dp2, dn2, pwT, pb, nwT, nb)
    if last:
        return pl.pallas_call(
            _layer_last_body,
            grid=(grid,),
            in_specs=in_specs,
            out_specs=pl.BlockSpec((1, 128), lambda i: (0, 0)),
            out_shape=jax.ShapeDtypeStruct((1, 128), jnp.float32),
        )(*args)
    return pl.pallas_call(
        _layer_body,
        grid=(grid,),
        in_specs=in_specs,
        out_specs=[
            pl.BlockSpec((BN, 128), lambda i: (i, 0)),
            pl.BlockSpec((4, BN, 64), lambda i: (0, i, 0)),
        ],
        out_shape=[
            jax.ShapeDtypeStruct((NP, 128), jnp.float32),
            jax.ShapeDtypeStruct((4, NP, 64), jnp.float32),
        ],
    )(*args)


def _dyn_body(x_r, cst_r, wqT_r, bq_r, wkvT_r, bkv_r, outwT_r, outb_r,
              n1g_r, n1b_r, ff1T_r, ff1b_r, ff2T_r, ff2b_r, n2g_r, n2b_r,
              lng_r, lnb_r, hm_r, z_r):
    x = x_r[...]
    cst = cst_r[...]
    emb_row = cst[0:1, :]
    hm = hm_r[...]
    hs = []
    for i in range(S):
        hs.append(x[:, i:i + 1] * emb_row + cst[1 + i:2 + i, :])
    q4 = _dot(hs[4], wqT_r[...]) + bq_r[...]
    wkvT = wkvT_r[...]
    bkv = bkv_r[...]
    kvs = [_dot(hs[i], wkvT) + bkv for i in range(S)]
    scal = 1.0 / np.sqrt(HD)
    sc = []
    for j in range(S):
        qk = q4 * kvs[j][:, :D]
        sc.append([jnp.sum(qk * hm[h:h + 1, :], axis=1, keepdims=True) * scal
                   for h in range(NH)])
    o4 = jnp.zeros_like(q4)
    for h in range(NH):
        m = sc[0][h]
        for j in range(1, S):
            m = jnp.maximum(m, sc[j][h])
        es = [jnp.exp(sc[j][h] - m) for j in range(S)]
        den = es[0]
        for j in range(1, S):
            den = den + es[j]
        hmask = hm[h:h + 1, :]
        for j in range(S):
            o4 = o4 + (es[j] / den) * (kvs[j][:, D:] * hmask)
    attn = _dot(o4, outwT_r[...]) + outb_r[...]
    h4 = _ln(hs[4] + attn, n1g_r[...], n1b_r[...])
    ff = _dot(jax.nn.relu(_dot(h4, ff1T_r[...]) + ff1b_r[...]), ff2T_r[...]) \
        + ff2b_r[...]
    h4 = _ln(h4 + ff, n2g_r[...], n2b_r[...])
    z_r[...] = _ln(h4, lng_r[...], lnb_r[...])


def _dynamics(xp, cst, wqT, bq, wkvT, bkv, outwT, outb, n1g, n1b,
              ff1T, ff1b, ff2T, ff2b, n2g, n2b, lng, lnb, hm):
    grid = NP // BN
    c = lambda shape: pl.BlockSpec(shape, lambda i: (0, 0))
    return pl.pallas_call(
        _dyn_body,
        grid=(grid,),
        in_specs=[
            pl.BlockSpec((BN, 128), lambda i: (i, 0)),
            c((8, 128)), c((128, 128)), c((1, 128)), c((128, 256)),
            c((1, 256)), c((128, 128)), c((1, 128)), c((1, 128)), c((1, 128)),
            c((128, 512)), c((1, 512)), c((512, 128)), c((1, 128)),
            c((1, 128)), c((1, 128)), c((1, 128)), c((1, 128)), c((4, 128)),
        ],
        out_specs=pl.BlockSpec((BN, 128), lambda i: (i, 0)),
        out_shape=jax.ShapeDtypeStruct((NP, 128), jnp.float32),
    )(xp, cst, wqT, bq, wkvT, bkv, outwT, outb, n1g, n1b,
      ff1T, ff1b, ff2T, ff2b, n2g, n2b, lng, lnb, hm)


def _modconst_body(sum_r, tg_r, tb_r, fwT_r, fb_r, flg_r, flb_r,
                   vwT_r, vb_r, aowT_r, aob_r, gb_r, c_r):
    g = sum_r[...] / float(N)
    eT = _ln(g, tg_r[...], tb_r[...])
    eT8 = jnp.broadcast_to(eT, (8, 128))
    film = _ln(_dot(eT8, fwT_r[...]) + fb_r[...], flg_r[...], flb_r[...])
    gb_r[...] = film
    c_r[...] = _dot(_dot(eT8, vwT_r[...]) + vb_r[...], aowT_r[...]) + aob_r[...]


def _modconst(psum, tg, tb, fwT, fb, flg, flb, vwT, vb, aowT, aob):
    c = lambda shape: pl.BlockSpec(shape, lambda: (0, 0))
    return pl.pallas_call(
        _modconst_body,
        in_specs=[c((1, 128)), c((1, 128)), c((1, 128)), c((128, 256)),
                  c((1, 256)), c((1, 256)), c((1, 256)), c((128, 128)),
                  c((1, 128)), c((128, 128)), c((1, 128))],
        out_specs=[c((8, 256)), c((8, 128))],
        out_shape=[jax.ShapeDtypeStruct((8, 256), jnp.float32),
                   jax.ShapeDtypeStruct((8, 128), jnp.float32)],
    )(psum, tg, tb, fwT, fb, flg, flb, vwT, vb, aowT, aob)


def _modk_body(z_r, gb_r, c_r, ang_r, anb_r, f1T_r, f2T_r, fb_r,
               ssum_r, smax_r):
    i = pl.program_id(0)
    z = z_r[...]
    gamma = gb_r[0:1, :128]
    beta = gb_r[0:1, 128:]
    cvec = c_r[0:1, :]
    mg = gamma * z + beta
    ma = _ln(cvec + z, ang_r[...], anb_r[...])
    zs = _dot(mg, f1T_r[...]) + _dot(ma, f2T_r[...]) + fb_r[...]
    ridx = lax.broadcasted_iota(jnp.int32, (BN, 1), 0) + i * BN
    valid = ridx < N
    psum = jnp.sum(jnp.where(valid, zs, 0.0), axis=0, keepdims=True)
    pmax = jnp.max(jnp.where(valid, zs, -1e30), axis=0, keepdims=True)
    prev_s = jnp.where(i == 0, jnp.zeros((1, D), jnp.float32), ssum_r[...])
    prev_m = jnp.where(i == 0, jnp.full((1, D), -1e30, jnp.float32),
                       smax_r[...])
    ssum_r[...] = prev_s + psum
    smax_r[...] = jnp.maximum(prev_m, pmax)


def _modk(z, gb, c8, ang, anb, f1T, f2T, fb):
    grid = NP // BN
    c = lambda shape: pl.BlockSpec(shape, lambda i: (0, 0))
    return pl.pallas_call(
        _modk_body,
        grid=(grid,),
        in_specs=[pl.BlockSpec((BN, 128), lambda i: (i, 0)),
                  c((8, 256)), c((8, 128)), c((1, 128)), c((1, 128)),
                  c((128, 128)), c((128, 128)), c((1, 128))],
        out_specs=[c((1, 128)), c((1, 128))],
        out_shape=[jax.ShapeDtypeStruct((1, 128), jnp.float32)] * 2,
    )(z, gb, c8, ang, anb, f1T, f2T, fb)


def _final_body(ssum_r, smax_r, awT_r, ab_r, mwT_r, mb_r, fckT_r, fckb_r,
                fcyw_r, fcyb_r, k_r, y_r):
    ssum = ssum_r[...]
    za8 = jnp.broadcast_to(ssum / float(N), (8, 128))
    zm8 = jnp.broadcast_to(smax_r[...], (8, 128))
    logit = _dot(za8, awT_r[...]) + ab_r[...] + _dot(zm8, mwT_r[...]) + mb_r[...]
    att = jax.nn.sigmoid(logit)
    eg = att * jnp.broadcast_to(ssum, (8, 128))
    k8 = _dot(eg, fckT_r[...])
    k = k8[0:1, 0:1] + fckb_r[...]
    k_r[...] = k
    y_r[...] = jax.nn.sigmoid(k * fcyw_r[...] + fcyb_r[...])


def _final(ssum, smax, awT, ab, mwT, mb, fckT, fckb, fcyw, fcyb):
    c = lambda shape: pl.BlockSpec(shape, lambda: (0, 0))
    return pl.pallas_call(
        _final_body,
        in_specs=[c((1, 128)), c((1, 128)), c((128, 128)), c((1, 128)),
                  c((128, 128)), c((1, 128)), c((128, 128)), c((1, 1)),
                  c((1, 1)), c((1, 1))],
        out_specs=[c((1, 1)), c((1, 1))],
        out_shape=[jax.ShapeDtypeStruct((1, 1), jnp.float32)] * 2,
    )(ssum, smax, awT, ab, mwT, mb, fckT, fckb, fcyw, fcyb)


def _pe_rows():
    pos = np.arange(S)[:, None].astype(np.float32)
    div = np.exp(np.arange(0, D, 2).astype(np.float32) * (-np.log(10000.0) / D))
    pe = np.zeros((S, D), dtype=np.float32)
    pe[:, 0::2] = np.sin(pos * div)
    pe[:, 1::2] = np.cos(pos * div)
    return pe


def kernel(x_dyn, x_topo, edge_weight, params, edge_index, batch):
    r2 = lambda v: v.reshape(1, -1)

    p = params["dyn"]
    x5 = x_dyn.reshape(N, S)
    xp = jnp.zeros((NP, 128), jnp.float32).at[:N, :S].set(x5)
    pe = _pe_rows()
    cst = jnp.zeros((8, 128), jnp.float32)
    cst = cst.at[0].set(p["emb_w"][:, 0])
    cst = cst.at[1:1 + S].set(p["emb_b"][None, :] + pe)
    qkv_w, qkv_b = p["qkv_w"][0], p["qkv_b"][0]
    wqT = qkv_w[:D].T
    bq = r2(qkv_b[:D])
    wkvT = qkv_w[D:].T
    bkv = r2(qkv_b[D:])
    hm = jnp.asarray(np.repeat(np.eye(NH, dtype=np.float32), HD, axis=1))
    Z = _dynamics(xp, cst, wqT, bq, wkvT, bkv, p["out_w"][0].T,
                  r2(p["out_b"][0]), r2(p["n1_g"][0]), r2(p["n1_b"][0]),
                  p["ff1_w"][0].T, r2(p["ff1_b"][0]), p["ff2_w"][0].T,
                  r2(p["ff2_b"][0]), r2(p["n2_g"][0]), r2(p["n2_b"][0]),
                  r2(p["ln_g"]), r2(p["ln_b"]), hm)

    row = edge_index[0].reshape(2500, 128)
    col = edge_index[1].reshape(2500, 128)
    ew = edge_weight.reshape(2500, 128)
    g0, g1, tcol = _edgeprep(row, col, ew)
    pad = ((0, 0), (0, EP - ET))
    tcol_sc = jnp.pad(tcol.reshape(NS, ET), pad, constant_values=TRASH)
    tcol_sc = tcol_sc.reshape(NS, NCHUNK, CH)
    gidx_sc = jnp.stack([
        jnp.pad(g0.reshape(NS, ET), pad).reshape(NS, NCHUNK, CH),
        jnp.pad(g1.reshape(NS, ET), pad).reshape(NS, NCHUNK, CH),
    ])

    degparts = _degree_sc(tcol_sc)
    dinv_t, dsq_t = _dinv_tc(degparts[0].reshape(162, 128))
    dinv = dinv_t.reshape(ACC_ROWS)[:2 * NP]
    dsq = dsq_t.reshape(ACC_ROWS)[:2 * NP]
    dinvp = dinv[:NP].reshape(NP, 1)
    dinvn = dinv[NP:].reshape(NP, 1)
    dp2 = dsq[:NP].reshape(NP, 1)
    dn2 = dsq[NP:].reshape(NP, 1)

    t = params["topo"]
    h = jnp.zeros((NP, 128), jnp.float32).at[:N].set(x_topo)
    hs = _hsbuild(h, dinvp, dinvn)
    for l in range(TNL):
        agg = _agg_sc(hs.reshape(4 * NP, 64), gidx_sc, tcol_sc)
        out = _gcn_layer(h, agg, dinvp, dinvn, dp2, dn2,
                         t["pw"][l].T, r2(t["pb"][l]),
                         t["nw"][l].T, r2(t["nb"][l]), last=(l == TNL - 1))
        if l < TNL - 1:
            h, hs = out
        else:
            pooled = out

    mo = params["mod"]
    vb = jnp.split(mo["in_b"], 3)[2]
    gb8, c8 = _modconst(pooled, r2(t["ln_g"]), r2(t["ln_b"]),
                        mo["film_w"].T, r2(mo["film_b"]),
                        r2(mo["film_ln_g"]), r2(mo["film_ln_b"]),
                        mo["v_w"].T, r2(vb), mo["ao_w"].T, r2(mo["ao_b"]))

    ssum, smax = _modk(Z, gb8, c8, r2(mo["an_g"]), r2(mo["an_b"]),
                       mo["fus_w"][:, :D].T, mo["fus_w"][:, D:].T,
                       r2(mo["fus_b"]))

    kp, pr = params["ksp"], params["pred"]
    fckT = jnp.zeros((128, 128), jnp.float32).at[:, 0].set(pr["fck_w"][0])
    k, y = _final(ssum, smax, kp["avg_w"].T, r2(kp["avg_b"]),
                  kp["max_w"].T, r2(kp["max_b"]), fckT,
                  pr["fck_b"].reshape(1, 1), pr["fcy_w"].reshape(1, 1),
                  pr["fcy_b"].reshape(1, 1))
    return k, y
